# baseline (speedup 1.0000x reference)
import functools
import math

import jax
import jax.numpy as jnp
from jax import lax
from jax.experimental import pallas as pl
from jax.experimental.pallas import tpu as pltpu
from jax.experimental.pallas import tpu_sc as plsc

N = 100000
E = 1600000
IN = 6
D1 = 32
G = 64
BN_EPS = 1e-5

NC = 2
NS = 16
LANES = 16

NPAD = 100352
HALF = NPAD // NC
TILE_ROWS = HALF // NS
TRASH = HALF
ACC_ROWS = HALF + 8
CH = 128
IDXB = 384
KCH = IDXB // CH
NB = 2 * (-(-E // (2 * NS * IDXB)))
EPT = NB * IDXB
EPAD = NS * EPT
EALLOC = NS * (NB + 2) * IDXB
NPAIR = NB // 2
DSTPAD = 4 * HALF

R = 6272
GRID = NPAD // R
INV_S = 1.0 / math.sqrt(1.0 + BN_EPS)


def _make_sc_agg(D):
    mesh = plsc.VectorSubcoreMesh(core_axis_name="c", subcore_axis_name="s")

    @functools.partial(
        pl.kernel,
        out_type=jax.ShapeDtypeStruct((NPAD, D), jnp.float32),
        mesh=mesh,
        scratch_types=[
            pltpu.VMEM((2, IDXB), jnp.int32),
            pltpu.VMEM((2, IDXB), jnp.int32),
            pltpu.VMEM((IDXB,), jnp.int32),
            pltpu.VMEM((IDXB,), jnp.int32),
            pltpu.VMEM((IDXB, D), jnp.float32),
            pltpu.VMEM((IDXB, D), jnp.float32),
            pltpu.VMEM_SHARED((ACC_ROWS, D), jnp.float32),
            pltpu.SemaphoreType.DMA,
            pltpu.SemaphoreType.DMA,
            pltpu.SemaphoreType.DMA,
            pltpu.SemaphoreType.DMA,
        ],
        compiler_params=pltpu.CompilerParams(use_tc_tiling_on_sc=False),
    )
    def agg(x_hbm, epk_hbm, out_hbm, eidx0, eidx1,
            d20, d21, rows0, rows1, acc, semg0, semg1, semi0, semi1):
        c = lax.axis_index("c")
        s = lax.axis_index("s")
        base = c * HALF
        gr = base + s * TILE_ROWS
        lr = s * TILE_ROWS
        pltpu.sync_copy(x_hbm.at[pl.ds(gr, TILE_ROWS)],
                        acc.at[pl.ds(lr, TILE_ROWS)])
        plsc.subcore_barrier()

        bbase = s * (NB + 2)
        eidx = (eidx0, eidx1)
        d2 = (d20, d21)
        rows = (rows0, rows1)
        semg = (semg0, semg1)
        semi = (semi0, semi1)

        def load_idx(i, p):
            pltpu.async_copy(epk_hbm.at[bbase + i], eidx[p], semi[p])

        def wait_idx(p):
            pltpu.make_async_copy(epk_hbm.at[0], eidx[p], semi[p]).wait()

        def remap(p):
            for j in range(IDXB // LANES):
                d = eidx[p][1, pl.ds(j * LANES, LANES)]
                loc = d - base
                ok = loc.astype(jnp.uint32) < jnp.uint32(HALF)
                d2[p][pl.ds(j * LANES, LANES)] = jnp.where(ok, loc, TRASH)

        def fire_gathers(p):
            pltpu.async_copy(x_hbm.at[eidx[p].at[0]], rows[p], semg[p])

        def wait_gathers(p):
            pltpu.make_async_copy(x_hbm.at[eidx[p].at[0]], rows[p],
                                  semg[p]).wait()

        def scatter(p):
            pltpu.sync_copy(rows[p], acc.at[d2[p]], add=True)

        load_idx(0, 0)
        wait_idx(0)
        remap(0)
        fire_gathers(0)
        load_idx(1, 1)

        def pair(t, carry):
            i0 = 2 * t
            wait_idx(1)
            remap(1)
            wait_gathers(0)
            fire_gathers(1)
            load_idx(i0 + 2, 0)
            scatter(0)
            wait_idx(0)
            remap(0)
            wait_gathers(1)
            fire_gathers(0)
            load_idx(i0 + 3, 1)
            scatter(1)
            return carry

        lax.fori_loop(0, NPAIR, pair, 0)
        wait_idx(1)
        wait_gathers(0)

        plsc.subcore_barrier()
        pltpu.sync_copy(acc.at[pl.ds(lr, TILE_ROWS)],
                        out_hbm.at[pl.ds(gr, TILE_ROWS)])

    return agg


_sc_agg8 = _make_sc_agg(8)
_sc_agg32 = _make_sc_agg(D1)


def _mlp_body(h_ref, wa_ref, ba_ref, wb_ref, bb_ref, g_ref, be_ref, o_ref):
    hb = h_ref[...]
    t = jnp.maximum(
        jnp.dot(hb, wa_ref[...], preferred_element_type=jnp.float32)
        + ba_ref[...], 0.0)
    u = (jnp.dot(t, wb_ref[...], preferred_element_type=jnp.float32)
         + bb_ref[...])
    v = jnp.maximum(u, 0.0)
    o_ref[...] = v * (g_ref[...] * INV_S) + be_ref[...]


def _mlp(h, wa, ba, wb, bb, g, be):
    din = h.shape[1]
    return pl.pallas_call(
        _mlp_body,
        grid=(GRID,),
        in_specs=[
            pl.BlockSpec((R, din), lambda i: (i, 0)),
            pl.BlockSpec((din, D1), lambda i: (0, 0)),
            pl.BlockSpec((1, D1), lambda i: (0, 0)),
            pl.BlockSpec((D1, D1), lambda i: (0, 0)),
            pl.BlockSpec((1, D1), lambda i: (0, 0)),
            pl.BlockSpec((1, D1), lambda i: (0, 0)),
            pl.BlockSpec((1, D1), lambda i: (0, 0)),
        ],
        out_specs=pl.BlockSpec((R, D1), lambda i: (i, 0)),
        out_shape=jax.ShapeDtypeStruct((NPAD, D1), jnp.float32),
    )(h, wa, ba.reshape(1, D1), wb, bb.reshape(1, D1),
      g.reshape(1, D1), be.reshape(1, D1))


def _mlp3_pool_head_body(h_ref, wa_ref, ba_ref, wb_ref, bb_ref, g_ref, be_ref,
                         ids_ref, wlb_ref, blb_ref, wlm_ref, blm_ref,
                         o_ref, maxtab):
    i = pl.program_id(0)

    @pl.when(i == 0)
    def _():
        maxtab[...] = jnp.full((G, D1), -jnp.inf, jnp.float32)

    hb = h_ref[...]
    t = jnp.maximum(
        jnp.dot(hb, wa_ref[...], preferred_element_type=jnp.float32)
        + ba_ref[...], 0.0)
    u = (jnp.dot(t, wb_ref[...], preferred_element_type=jnp.float32)
         + bb_ref[...])
    v = jnp.maximum(u, 0.0)
    x3 = v * (g_ref[...] * INV_S) + be_ref[...]

    ids = ids_ref[...]
    lo = ids[0, 0]
    hi = jnp.minimum(ids[R - 1, 0], G - 1)

    def seg(gidx, carry):
        m = ids == gidx
        pm = jnp.max(jnp.where(m, x3, -jnp.inf), axis=0, keepdims=True)
        maxtab[pl.ds(gidx, 1), :] = jnp.maximum(maxtab[pl.ds(gidx, 1), :], pm)
        return carry

    lax.fori_loop(lo, hi + 1, seg, 0)

    @pl.when(i == pl.num_programs(0) - 1)
    def _():
        emb = maxtab[...]
        hh = jnp.maximum(
            jnp.dot(emb, wlb_ref[...], preferred_element_type=jnp.float32)
            + blb_ref[...], 0.0)
        logit = (jnp.dot(hh, wlm_ref[...], preferred_element_type=jnp.float32)
                 + blm_ref[...])
        o_ref[...] = 1.0 / (1.0 + jnp.exp(-logit))


def _mlp3_pool_head(h, wa, ba, wb, bb, g, be, ids, wlb, blb, wlm, blm):
    return pl.pallas_call(
        _mlp3_pool_head_body,
        grid=(GRID,),
        in_specs=[
            pl.BlockSpec((R, D1), lambda i: (i, 0)),
            pl.BlockSpec((D1, D1), lambda i: (0, 0)),
            pl.BlockSpec((1, D1), lambda i: (0, 0)),
            pl.BlockSpec((D1, D1), lambda i: (0, 0)),
            pl.BlockSpec((1, D1), lambda i: (0, 0)),
            pl.BlockSpec((1, D1), lambda i: (0, 0)),
            pl.BlockSpec((1, D1), lambda i: (0, 0)),
            pl.BlockSpec((R, 1), lambda i: (i, 0)),
            pl.BlockSpec((D1, 16), lambda i: (0, 0)),
            pl.BlockSpec((1, 16), lambda i: (0, 0)),
            pl.BlockSpec((16, 1), lambda i: (0, 0)),
            pl.BlockSpec((1, 1), lambda i: (0, 0)),
        ],
        out_specs=pl.BlockSpec((G, 1), lambda i: (0, 0)),
        out_shape=jax.ShapeDtypeStruct((G, 1), jnp.float32),
        scratch_shapes=[pltpu.VMEM((G, D1), jnp.float32)],
    )(h, wa, ba.reshape(1, D1), wb, bb.reshape(1, D1),
      g.reshape(1, D1), be.reshape(1, D1), ids,
      wlb, blb.reshape(1, 16), wlm, blm.reshape(1, 1))


def kernel(data, edge_index, batch, W1a, b1a, W1b, b1b, W2a, b2a, W2b, b2b,
           W3a, b3a, W3b, b3b, g1, be1, g2, be2, g3, be3, Wlb, blb, Wlm, blm):
    src = edge_index[0]
    dst = edge_index[1]
    srcp = jnp.concatenate(
        [src, jnp.zeros((EPAD - E,), jnp.int32)]).reshape(NS, NB, 1, IDXB)
    dstp = jnp.concatenate(
        [dst, jnp.full((EPAD - E,), DSTPAD, jnp.int32)]).reshape(
            NS, NB, 1, IDXB)
    slack = jnp.broadcast_to(
        jnp.stack([jnp.zeros((IDXB,), jnp.int32),
                   jnp.full((IDXB,), DSTPAD, jnp.int32)]), (NS, 2, 2, IDXB))
    epk = jnp.concatenate(
        [jnp.concatenate([srcp, dstp], axis=2), slack],
        axis=1).reshape(NS * (NB + 2), 2, IDXB)
    x0 = jnp.pad(data, ((0, NPAD - N), (0, 8 - IN)))
    w1a_p = jnp.pad(W1a, ((0, 8 - IN), (0, 0)))
    ids = jnp.concatenate(
        [batch, jnp.full((NPAD - N,), 127, jnp.int32)]).reshape(NPAD, 1)

    h1 = _sc_agg8(x0, epk)
    x1 = _mlp(h1, w1a_p, b1a, W1b, b1b, g1, be1)
    h2 = _sc_agg32(x1, epk)
    x2 = _mlp(h2, W2a, b2a, W2b, b2b, g2, be2)
    h3 = _sc_agg32(x2, epk)
    return _mlp3_pool_head(h3, W3a, b3a, W3b, b3b, g3, be3, ids,
                           Wlb, blb, Wlm, blm)


# speedup vs baseline: 1.0003x; 1.0003x over previous
import functools
import math

import jax
import jax.numpy as jnp
from jax import lax
from jax.experimental import pallas as pl
from jax.experimental.pallas import tpu as pltpu
from jax.experimental.pallas import tpu_sc as plsc

N = 100000
E = 1600000
IN = 6
D1 = 32
G = 64
BN_EPS = 1e-5

NC = 2
NS = 16
LANES = 16

NPAD = 100352
HALF = NPAD // NC
TILE_ROWS = HALF // NS
TRASH = HALF
ACC_ROWS = HALF + 8
CH = 128
IDXB = 384
KCH = IDXB // CH
NB = 2 * (-(-E // (2 * NS * IDXB)))
EPT = NB * IDXB
EPAD = NS * EPT
EALLOC = NS * (NB + 2) * IDXB
NPAIR = NB // 2
DSTPAD = 4 * HALF

R = 6272
GRID = NPAD // R
INV_S = 1.0 / math.sqrt(1.0 + BN_EPS)


def _make_sc_agg(D):
    mesh = plsc.VectorSubcoreMesh(core_axis_name="c", subcore_axis_name="s")

    @functools.partial(
        pl.kernel,
        out_type=jax.ShapeDtypeStruct((NPAD, D), jnp.float32),
        mesh=mesh,
        scratch_types=[
            pltpu.VMEM((2, IDXB), jnp.int32),
            pltpu.VMEM((2, IDXB), jnp.int32),
            pltpu.VMEM((IDXB,), jnp.int32),
            pltpu.VMEM((IDXB,), jnp.int32),
            pltpu.VMEM((IDXB, D), jnp.float32),
            pltpu.VMEM((IDXB, D), jnp.float32),
            pltpu.VMEM_SHARED((ACC_ROWS, D), jnp.float32),
            pltpu.SemaphoreType.DMA,
            pltpu.SemaphoreType.DMA,
            pltpu.SemaphoreType.DMA,
            pltpu.SemaphoreType.DMA,
        ],
        compiler_params=pltpu.CompilerParams(use_tc_tiling_on_sc=False),
    )
    def agg(x_hbm, epk_hbm, out_hbm, eidx0, eidx1,
            d20, d21, rows0, rows1, acc, semg0, semg1, semi0, semi1):
        c = lax.axis_index("c")
        s = lax.axis_index("s")
        base = c * HALF
        gr = base + s * TILE_ROWS
        lr = s * TILE_ROWS
        pltpu.sync_copy(x_hbm.at[pl.ds(gr, TILE_ROWS)],
                        acc.at[pl.ds(lr, TILE_ROWS)])
        plsc.subcore_barrier()

        bbase = s * (NB + 2)
        eidx = (eidx0, eidx1)
        d2 = (d20, d21)
        rows = (rows0, rows1)
        semg = (semg0, semg1)
        semi = (semi0, semi1)

        def load_idx(i, p):
            pltpu.async_copy(epk_hbm.at[bbase + i], eidx[p], semi[p])

        def wait_idx(p):
            pltpu.make_async_copy(epk_hbm.at[0], eidx[p], semi[p]).wait()

        def remap(p):
            for j in range(IDXB // LANES):
                d = eidx[p][1, pl.ds(j * LANES, LANES)]
                loc = d - base
                ok = loc.astype(jnp.uint32) < jnp.uint32(HALF)
                d2[p][pl.ds(j * LANES, LANES)] = jnp.where(ok, loc, TRASH)

        def fire_gathers(p):
            for k in range(KCH):
                pltpu.async_copy(
                    x_hbm.at[eidx[p].at[0].at[pl.ds(k * CH, CH)]],
                    rows[p].at[pl.ds(k * CH, CH)], semg[p])

        def wait_gathers(p):
            for k in range(KCH):
                pltpu.make_async_copy(
                    x_hbm.at[eidx[p].at[0].at[pl.ds(k * CH, CH)]],
                    rows[p].at[pl.ds(k * CH, CH)], semg[p]).wait()

        def scatter(p):
            for k in range(KCH):
                pltpu.sync_copy(rows[p].at[pl.ds(k * CH, CH)],
                                acc.at[d2[p].at[pl.ds(k * CH, CH)]], add=True)

        load_idx(0, 0)
        wait_idx(0)
        remap(0)
        fire_gathers(0)
        load_idx(1, 1)

        def pair(t, carry):
            i0 = 2 * t
            wait_idx(1)
            remap(1)
            wait_gathers(0)
            fire_gathers(1)
            load_idx(i0 + 2, 0)
            scatter(0)
            wait_idx(0)
            remap(0)
            wait_gathers(1)
            fire_gathers(0)
            load_idx(i0 + 3, 1)
            scatter(1)
            return carry

        lax.fori_loop(0, NPAIR, pair, 0)
        wait_idx(1)
        wait_gathers(0)

        plsc.subcore_barrier()
        pltpu.sync_copy(acc.at[pl.ds(lr, TILE_ROWS)],
                        out_hbm.at[pl.ds(gr, TILE_ROWS)])

    return agg


_sc_agg8 = _make_sc_agg(8)
_sc_agg32 = _make_sc_agg(D1)


def _mlp_body(h_ref, wa_ref, ba_ref, wb_ref, bb_ref, g_ref, be_ref, o_ref):
    hb = h_ref[...]
    t = jnp.maximum(
        jnp.dot(hb, wa_ref[...], preferred_element_type=jnp.float32)
        + ba_ref[...], 0.0)
    u = (jnp.dot(t, wb_ref[...], preferred_element_type=jnp.float32)
         + bb_ref[...])
    v = jnp.maximum(u, 0.0)
    o_ref[...] = v * (g_ref[...] * INV_S) + be_ref[...]


def _mlp(h, wa, ba, wb, bb, g, be):
    din = h.shape[1]
    return pl.pallas_call(
        _mlp_body,
        grid=(GRID,),
        in_specs=[
            pl.BlockSpec((R, din), lambda i: (i, 0)),
            pl.BlockSpec((din, D1), lambda i: (0, 0)),
            pl.BlockSpec((1, D1), lambda i: (0, 0)),
            pl.BlockSpec((D1, D1), lambda i: (0, 0)),
            pl.BlockSpec((1, D1), lambda i: (0, 0)),
            pl.BlockSpec((1, D1), lambda i: (0, 0)),
            pl.BlockSpec((1, D1), lambda i: (0, 0)),
        ],
        out_specs=pl.BlockSpec((R, D1), lambda i: (i, 0)),
        out_shape=jax.ShapeDtypeStruct((NPAD, D1), jnp.float32),
    )(h, wa, ba.reshape(1, D1), wb, bb.reshape(1, D1),
      g.reshape(1, D1), be.reshape(1, D1))


def _mlp3_pool_head_body(h_ref, wa_ref, ba_ref, wb_ref, bb_ref, g_ref, be_ref,
                         ids_ref, wlb_ref, blb_ref, wlm_ref, blm_ref,
                         o_ref, maxtab):
    i = pl.program_id(0)

    @pl.when(i == 0)
    def _():
        maxtab[...] = jnp.full((G, D1), -jnp.inf, jnp.float32)

    hb = h_ref[...]
    t = jnp.maximum(
        jnp.dot(hb, wa_ref[...], preferred_element_type=jnp.float32)
        + ba_ref[...], 0.0)
    u = (jnp.dot(t, wb_ref[...], preferred_element_type=jnp.float32)
         + bb_ref[...])
    v = jnp.maximum(u, 0.0)
    x3 = v * (g_ref[...] * INV_S) + be_ref[...]

    ids = ids_ref[...]
    lo = ids[0, 0]
    hi = jnp.minimum(ids[R - 1, 0], G - 1)

    def seg(gidx, carry):
        m = ids == gidx
        pm = jnp.max(jnp.where(m, x3, -jnp.inf), axis=0, keepdims=True)
        maxtab[pl.ds(gidx, 1), :] = jnp.maximum(maxtab[pl.ds(gidx, 1), :], pm)
        return carry

    lax.fori_loop(lo, hi + 1, seg, 0)

    @pl.when(i == pl.num_programs(0) - 1)
    def _():
        emb = maxtab[...]
        hh = jnp.maximum(
            jnp.dot(emb, wlb_ref[...], preferred_element_type=jnp.float32)
            + blb_ref[...], 0.0)
        logit = (jnp.dot(hh, wlm_ref[...], preferred_element_type=jnp.float32)
                 + blm_ref[...])
        o_ref[...] = 1.0 / (1.0 + jnp.exp(-logit))


def _mlp3_pool_head(h, wa, ba, wb, bb, g, be, ids, wlb, blb, wlm, blm):
    return pl.pallas_call(
        _mlp3_pool_head_body,
        grid=(GRID,),
        in_specs=[
            pl.BlockSpec((R, D1), lambda i: (i, 0)),
            pl.BlockSpec((D1, D1), lambda i: (0, 0)),
            pl.BlockSpec((1, D1), lambda i: (0, 0)),
            pl.BlockSpec((D1, D1), lambda i: (0, 0)),
            pl.BlockSpec((1, D1), lambda i: (0, 0)),
            pl.BlockSpec((1, D1), lambda i: (0, 0)),
            pl.BlockSpec((1, D1), lambda i: (0, 0)),
            pl.BlockSpec((R, 1), lambda i: (i, 0)),
            pl.BlockSpec((D1, 16), lambda i: (0, 0)),
            pl.BlockSpec((1, 16), lambda i: (0, 0)),
            pl.BlockSpec((16, 1), lambda i: (0, 0)),
            pl.BlockSpec((1, 1), lambda i: (0, 0)),
        ],
        out_specs=pl.BlockSpec((G, 1), lambda i: (0, 0)),
        out_shape=jax.ShapeDtypeStruct((G, 1), jnp.float32),
        scratch_shapes=[pltpu.VMEM((G, D1), jnp.float32)],
    )(h, wa, ba.reshape(1, D1), wb, bb.reshape(1, D1),
      g.reshape(1, D1), be.reshape(1, D1), ids,
      wlb, blb.reshape(1, 16), wlm, blm.reshape(1, 1))


def kernel(data, edge_index, batch, W1a, b1a, W1b, b1b, W2a, b2a, W2b, b2b,
           W3a, b3a, W3b, b3b, g1, be1, g2, be2, g3, be3, Wlb, blb, Wlm, blm):
    src = edge_index[0]
    dst = edge_index[1]
    srcp = jnp.concatenate(
        [src, jnp.zeros((EPAD - E,), jnp.int32)]).reshape(NS, NB, 1, IDXB)
    dstp = jnp.concatenate(
        [dst, jnp.full((EPAD - E,), DSTPAD, jnp.int32)]).reshape(
            NS, NB, 1, IDXB)
    slack = jnp.broadcast_to(
        jnp.stack([jnp.zeros((IDXB,), jnp.int32),
                   jnp.full((IDXB,), DSTPAD, jnp.int32)]), (NS, 2, 2, IDXB))
    epk = jnp.concatenate(
        [jnp.concatenate([srcp, dstp], axis=2), slack],
        axis=1).reshape(NS * (NB + 2), 2, IDXB)
    x0 = jnp.pad(data, ((0, NPAD - N), (0, 8 - IN)))
    w1a_p = jnp.pad(W1a, ((0, 8 - IN), (0, 0)))
    ids = jnp.concatenate(
        [batch, jnp.full((NPAD - N,), 127, jnp.int32)]).reshape(NPAD, 1)

    h1 = _sc_agg8(x0, epk)
    x1 = _mlp(h1, w1a_p, b1a, W1b, b1b, g1, be1)
    h2 = _sc_agg32(x1, epk)
    x2 = _mlp(h2, W2a, b2a, W2b, b2b, g2, be2)
    h3 = _sc_agg32(x2, epk)
    return _mlp3_pool_head(h3, W3a, b3a, W3b, b3b, g3, be3, ids,
                           Wlb, blb, Wlm, blm)


# speedup vs baseline: 1.7945x; 1.7940x over previous
import functools
import math

import jax
import jax.numpy as jnp
from jax import lax
from jax.experimental import pallas as pl
from jax.experimental.pallas import tpu as pltpu
from jax.experimental.pallas import tpu_sc as plsc

N = 100000
E = 1600000
IN = 6
D1 = 32
G = 64
BN_EPS = 1e-5

NC = 2
NS = 16
LANES = 16

NPAD = 100352
HALF = NPAD // NC
TILE_ROWS = HALF // NS
TRASH = HALF
ACC_ROWS = HALF + NS
CH = 128
IDXB = 384
KCH = IDXB // CH
NB = 2 * (-(-E // (2 * NS * IDXB)))
EPT = NB * IDXB
EPAD = NS * EPT
EALLOC = NS * (NB + 2) * IDXB
NPAIR = NB // 2
DSTPAD = 4 * HALF

R = 6272
GRID = NPAD // R
INV_S = 1.0 / math.sqrt(1.0 + BN_EPS)


def _make_sc_agg(D):
    mesh = plsc.VectorSubcoreMesh(core_axis_name="c", subcore_axis_name="s")

    @functools.partial(
        pl.kernel,
        out_type=jax.ShapeDtypeStruct((NPAD, D), jnp.float32),
        mesh=mesh,
        scratch_types=[
            pltpu.VMEM((2, IDXB), jnp.int32),
            pltpu.VMEM((2, IDXB), jnp.int32),
            pltpu.VMEM((KCH, CH), jnp.int32),
            pltpu.VMEM((KCH, CH), jnp.int32),
            pltpu.VMEM((IDXB, D), jnp.float32),
            pltpu.VMEM((IDXB, D), jnp.float32),
            pltpu.VMEM_SHARED((ACC_ROWS, D), jnp.float32),
            pltpu.SemaphoreType.DMA,
            pltpu.SemaphoreType.DMA,
            pltpu.SemaphoreType.DMA,
            pltpu.SemaphoreType.DMA,
        ],
        compiler_params=pltpu.CompilerParams(use_tc_tiling_on_sc=False),
    )
    def agg(x_hbm, epk_hbm, out_hbm, eidx0, eidx1,
            d20, d21, rows0, rows1, acc, semg0, semg1, semi0, semi1):
        c = lax.axis_index("c")
        s = lax.axis_index("s")
        base = c * HALF
        gr = base + s * TILE_ROWS
        lr = s * TILE_ROWS
        pltpu.sync_copy(x_hbm.at[pl.ds(gr, TILE_ROWS)],
                        acc.at[pl.ds(lr, TILE_ROWS)])
        plsc.subcore_barrier()

        bbase = s * (NB + 2)
        eidx = (eidx0, eidx1)
        d2 = (d20, d21)
        rows = (rows0, rows1)
        semg = (semg0, semg1)
        semi = (semi0, semi1)

        def load_idx(i, p):
            pltpu.async_copy(epk_hbm.at[bbase + i], eidx[p], semi[p])

        def wait_idx(p):
            pltpu.make_async_copy(epk_hbm.at[0], eidx[p], semi[p]).wait()

        trash = TRASH + s

        def remap(p):
            for j in range(IDXB // LANES):
                d = eidx[p][1, pl.ds(j * LANES, LANES)]
                loc = d - base
                ok = loc.astype(jnp.uint32) < jnp.uint32(HALF)
                d2[p][j // (CH // LANES),
                      pl.ds((j % (CH // LANES)) * LANES, LANES)] = (
                          jnp.where(ok, loc, trash))

        def fire_gathers(p):
            for k in range(KCH):
                pltpu.async_copy(
                    x_hbm.at[eidx[p].at[0].at[pl.ds(k * CH, CH)]],
                    rows[p].at[pl.ds(k * CH, CH)], semg[p])

        def wait_gathers(p):
            for k in range(KCH):
                pltpu.make_async_copy(
                    x_hbm.at[eidx[p].at[0].at[pl.ds(k * CH, CH)]],
                    rows[p].at[pl.ds(k * CH, CH)], semg[p]).wait()

        def scatter(p):
            for k in range(KCH):
                pltpu.sync_copy(rows[p].at[pl.ds(k * CH, CH)],
                                acc.at[d2[p].at[k]], add=True)

        load_idx(0, 0)
        wait_idx(0)
        remap(0)
        fire_gathers(0)
        load_idx(1, 1)

        def pair(t, carry):
            i0 = 2 * t
            wait_idx(1)
            remap(1)
            wait_gathers(0)
            fire_gathers(1)
            load_idx(i0 + 2, 0)
            scatter(0)
            wait_idx(0)
            remap(0)
            wait_gathers(1)
            fire_gathers(0)
            load_idx(i0 + 3, 1)
            scatter(1)
            return carry

        lax.fori_loop(0, NPAIR, pair, 0)
        wait_idx(1)
        wait_gathers(0)

        plsc.subcore_barrier()
        pltpu.sync_copy(acc.at[pl.ds(lr, TILE_ROWS)],
                        out_hbm.at[pl.ds(gr, TILE_ROWS)])

    return agg


_sc_agg8 = _make_sc_agg(8)
_sc_agg32 = _make_sc_agg(D1)


def _mlp_body(h_ref, wa_ref, ba_ref, wb_ref, bb_ref, g_ref, be_ref, o_ref):
    hb = h_ref[...]
    t = jnp.maximum(
        jnp.dot(hb, wa_ref[...], preferred_element_type=jnp.float32)
        + ba_ref[...], 0.0)
    u = (jnp.dot(t, wb_ref[...], preferred_element_type=jnp.float32)
         + bb_ref[...])
    v = jnp.maximum(u, 0.0)
    o_ref[...] = v * (g_ref[...] * INV_S) + be_ref[...]


def _mlp(h, wa, ba, wb, bb, g, be):
    din = h.shape[1]
    return pl.pallas_call(
        _mlp_body,
        grid=(GRID,),
        in_specs=[
            pl.BlockSpec((R, din), lambda i: (i, 0)),
            pl.BlockSpec((din, D1), lambda i: (0, 0)),
            pl.BlockSpec((1, D1), lambda i: (0, 0)),
            pl.BlockSpec((D1, D1), lambda i: (0, 0)),
            pl.BlockSpec((1, D1), lambda i: (0, 0)),
            pl.BlockSpec((1, D1), lambda i: (0, 0)),
            pl.BlockSpec((1, D1), lambda i: (0, 0)),
        ],
        out_specs=pl.BlockSpec((R, D1), lambda i: (i, 0)),
        out_shape=jax.ShapeDtypeStruct((NPAD, D1), jnp.float32),
    )(h, wa, ba.reshape(1, D1), wb, bb.reshape(1, D1),
      g.reshape(1, D1), be.reshape(1, D1))


def _mlp3_pool_head_body(h_ref, wa_ref, ba_ref, wb_ref, bb_ref, g_ref, be_ref,
                         ids_ref, wlb_ref, blb_ref, wlm_ref, blm_ref,
                         o_ref, maxtab):
    i = pl.program_id(0)

    @pl.when(i == 0)
    def _():
        maxtab[...] = jnp.full((G, D1), -jnp.inf, jnp.float32)

    hb = h_ref[...]
    t = jnp.maximum(
        jnp.dot(hb, wa_ref[...], preferred_element_type=jnp.float32)
        + ba_ref[...], 0.0)
    u = (jnp.dot(t, wb_ref[...], preferred_element_type=jnp.float32)
         + bb_ref[...])
    v = jnp.maximum(u, 0.0)
    x3 = v * (g_ref[...] * INV_S) + be_ref[...]

    ids = ids_ref[...]
    lo = ids[0, 0]
    hi = jnp.minimum(ids[R - 1, 0], G - 1)

    def seg(gidx, carry):
        m = ids == gidx
        pm = jnp.max(jnp.where(m, x3, -jnp.inf), axis=0, keepdims=True)
        maxtab[pl.ds(gidx, 1), :] = jnp.maximum(maxtab[pl.ds(gidx, 1), :], pm)
        return carry

    lax.fori_loop(lo, hi + 1, seg, 0)

    @pl.when(i == pl.num_programs(0) - 1)
    def _():
        emb = maxtab[...]
        hh = jnp.maximum(
            jnp.dot(emb, wlb_ref[...], preferred_element_type=jnp.float32)
            + blb_ref[...], 0.0)
        logit = (jnp.dot(hh, wlm_ref[...], preferred_element_type=jnp.float32)
                 + blm_ref[...])
        o_ref[...] = 1.0 / (1.0 + jnp.exp(-logit))


def _mlp3_pool_head(h, wa, ba, wb, bb, g, be, ids, wlb, blb, wlm, blm):
    return pl.pallas_call(
        _mlp3_pool_head_body,
        grid=(GRID,),
        in_specs=[
            pl.BlockSpec((R, D1), lambda i: (i, 0)),
            pl.BlockSpec((D1, D1), lambda i: (0, 0)),
            pl.BlockSpec((1, D1), lambda i: (0, 0)),
            pl.BlockSpec((D1, D1), lambda i: (0, 0)),
            pl.BlockSpec((1, D1), lambda i: (0, 0)),
            pl.BlockSpec((1, D1), lambda i: (0, 0)),
            pl.BlockSpec((1, D1), lambda i: (0, 0)),
            pl.BlockSpec((R, 1), lambda i: (i, 0)),
            pl.BlockSpec((D1, 16), lambda i: (0, 0)),
            pl.BlockSpec((1, 16), lambda i: (0, 0)),
            pl.BlockSpec((16, 1), lambda i: (0, 0)),
            pl.BlockSpec((1, 1), lambda i: (0, 0)),
        ],
        out_specs=pl.BlockSpec((G, 1), lambda i: (0, 0)),
        out_shape=jax.ShapeDtypeStruct((G, 1), jnp.float32),
        scratch_shapes=[pltpu.VMEM((G, D1), jnp.float32)],
    )(h, wa, ba.reshape(1, D1), wb, bb.reshape(1, D1),
      g.reshape(1, D1), be.reshape(1, D1), ids,
      wlb, blb.reshape(1, 16), wlm, blm.reshape(1, 1))


def kernel(data, edge_index, batch, W1a, b1a, W1b, b1b, W2a, b2a, W2b, b2b,
           W3a, b3a, W3b, b3b, g1, be1, g2, be2, g3, be3, Wlb, blb, Wlm, blm):
    src = edge_index[0]
    dst = edge_index[1]
    srcp = jnp.concatenate(
        [src, jnp.zeros((EPAD - E,), jnp.int32)]).reshape(NS, NB, 1, IDXB)
    dstp = jnp.concatenate(
        [dst, jnp.full((EPAD - E,), DSTPAD, jnp.int32)]).reshape(
            NS, NB, 1, IDXB)
    slack = jnp.broadcast_to(
        jnp.stack([jnp.zeros((IDXB,), jnp.int32),
                   jnp.full((IDXB,), DSTPAD, jnp.int32)]), (NS, 2, 2, IDXB))
    epk = jnp.concatenate(
        [jnp.concatenate([srcp, dstp], axis=2), slack],
        axis=1).reshape(NS * (NB + 2), 2, IDXB)
    x0 = jnp.pad(data, ((0, NPAD - N), (0, 8 - IN)))
    w1a_p = jnp.pad(W1a, ((0, 8 - IN), (0, 0)))
    ids = jnp.concatenate(
        [batch, jnp.full((NPAD - N,), 127, jnp.int32)]).reshape(NPAD, 1)

    h1 = _sc_agg8(x0, epk)
    x1 = _mlp(h1, w1a_p, b1a, W1b, b1b, g1, be1)
    h2 = _sc_agg32(x1, epk)
    x2 = _mlp(h2, W2a, b2a, W2b, b2b, g2, be2)
    h3 = _sc_agg32(x2, epk)
    return _mlp3_pool_head(h3, W3a, b3a, W3b, b3b, g3, be3, ids,
                           Wlb, blb, Wlm, blm)
